# baseline (speedup 1.0000x reference)
import jax
import jax.numpy as jnp
from jax.experimental import pallas as pl
from jax.experimental.pallas import tpu as pltpu


def _mbuild_kernel(w1t_ref, w2t_ref, b1_ref, b2_ref, w1_ref, fc1b_ref,
                   om_ref, oc_ref, wc_scr, bm_scr, accm_scr, accc_scr):
    f32 = jnp.float32
    g = pl.program_id(0)

    @pl.when(g == 0)
    def _init():
        wc = jnp.zeros((7, 7, 128), f32)
        full = jnp.zeros((1, 128), f32)
        no_i0 = jnp.zeros((1, 128), f32)
        no_j0 = jnp.zeros((1, 128), f32)
        no_ij = jnp.zeros((1, 128), f32)
        for i in range(3):
            for j in range(3):
                w2ij = w2t_ref[3 * i + j]
                bij = jnp.dot(w1t_ref[...], w2ij,
                              preferred_element_type=f32)
                wc = wc + jnp.pad(bij.reshape(3, 3, 128),
                                  ((2 * i, 4 - 2 * i), (2 * j, 4 - 2 * j),
                                   (0, 0)))
                s2 = jnp.dot(b1_ref[...], w2ij,
                             preferred_element_type=f32)
                full = full + s2
                if i >= 1:
                    no_i0 = no_i0 + s2
                if j >= 1:
                    no_j0 = no_j0 + s2
                if i >= 1 and j >= 1:
                    no_ij = no_ij + s2
        wc_scr[...] = jnp.pad(wc.reshape(49, 128),
                              ((0, 7), (0, 0))).astype(jnp.bfloat16)
        sidx = jax.lax.broadcasted_iota(jnp.int32, (64, 128), 0)
        oh0 = (sidx // 8) == 0
        ow0 = (sidx % 8) == 0
        bm = jnp.where(oh0 & ow0, no_ij,
                       jnp.where(oh0, no_i0, jnp.where(ow0, no_j0, full)))
        bm_scr[...] = bm + b2_ref[...]
        accm_scr[...] = jnp.zeros(accm_scr.shape, f32)
        accc_scr[...] = fc1b_ref[...]

    w16 = w1_ref[...].astype(jnp.bfloat16)
    part = jnp.dot(wc_scr[...], w16,
                   preferred_element_type=f32)
    contrib = part[:49].reshape(49, 8, 512)
    for a in range(7):
        for b in range(7):
            padded = jnp.pad(contrib[a * 7 + b],
                             ((b // 4, 1 - b // 4), (0, 0)))
            idx = (pl.ds(g + a // 4, 1), a % 4, slice(None), b % 4,
                   slice(None))
            accm_scr[idx] = accm_scr[idx] + padded[None]

    acc = accc_scr[...]
    for sl in range(8):
        row = bm_scr[pl.ds(8 * g + sl, 1), :]
        acc = acc + jnp.dot(row, w1_ref[:, 512 * sl:512 * (sl + 1)],
                            preferred_element_type=f32)
    accc_scr[...] = acc

    @pl.when(g == 7)
    def _fin():
        om_ref[...] = accm_scr[...].astype(om_ref.dtype)
        oc_ref[...] = accc_scr[...]


def _net_kernel(x_ref, m_ref, c_ref, w2_ref, b2_ref, w3_ref, b3_ref, o_ref):
    h = jnp.dot(x_ref[...], m_ref[...], preferred_element_type=jnp.float32)
    h = jnp.maximum(h + c_ref[...], 0.0)
    h = jnp.dot(h.astype(jnp.bfloat16), w2_ref[...].astype(jnp.bfloat16),
                preferred_element_type=jnp.float32)
    h = jnp.maximum(h + b2_ref[...], 0.0)
    o = jnp.dot(h.astype(jnp.bfloat16), w3_ref[...].astype(jnp.bfloat16),
                preferred_element_type=jnp.float32)
    o_ref[...] = (o + b3_ref[...]).astype(o_ref.dtype)


def kernel(conv1_w, conv1_b, conv2_w, conv2_b, fc1_w, fc1_b, fc2_w, fc2_b,
           fc3_w, fc3_b, x_nchw):
    f32 = jnp.float32
    bf16 = jnp.bfloat16

    w1t = conv1_w.reshape(64, 9).T
    w2t = conv2_w.reshape(128, 64, 9).transpose(2, 1, 0)

    m_raster, c = pl.pallas_call(
        _mbuild_kernel,
        out_shape=(jax.ShapeDtypeStruct((9, 4, 9, 4, 512), bf16),
                   jax.ShapeDtypeStruct((1, 512), f32)),
        grid=(8,),
        in_specs=[
            pl.BlockSpec((9, 64), lambda g: (0, 0)),
            pl.BlockSpec((9, 64, 128), lambda g: (0, 0, 0)),
            pl.BlockSpec((1, 64), lambda g: (0, 0)),
            pl.BlockSpec((1, 128), lambda g: (0, 0)),
            pl.BlockSpec((128, 4096), lambda g: (0, g)),
            pl.BlockSpec((1, 512), lambda g: (0, 0)),
        ],
        out_specs=(
            pl.BlockSpec((9, 4, 9, 4, 512), lambda g: (0, 0, 0, 0, 0)),
            pl.BlockSpec((1, 512), lambda g: (0, 0)),
        ),
        scratch_shapes=[
            pltpu.VMEM((56, 128), bf16),
            pltpu.VMEM((64, 128), f32),
            pltpu.VMEM((9, 4, 9, 4, 512), f32),
            pltpu.VMEM((1, 512), f32),
        ],
        compiler_params=pltpu.CompilerParams(
            dimension_semantics=("arbitrary",),
            vmem_limit_bytes=64 * 1024 * 1024,
        ),
    )(w1t, w2t, conv1_b.reshape(1, 64), conv2_b.reshape(1, 128),
      fc1_w.reshape(128, 64 * 512), fc1_b.reshape(1, 512))
    M = m_raster.reshape(36 * 36, 512)

    B = x_nchw.shape[0]
    xp = jnp.pad(x_nchw.reshape(B, 32, 32).astype(bf16),
                 ((0, 0), (3, 1), (3, 1)))
    xf = xp.reshape(B, 1296)

    TB = 128
    Bp = (B + TB - 1) // TB * TB
    if Bp != B:
        xf = jnp.pad(xf, ((0, Bp - B), (0, 0)))

    out = pl.pallas_call(
        _net_kernel,
        out_shape=jax.ShapeDtypeStruct((Bp, 2), f32),
        grid=(Bp // TB,),
        in_specs=[
            pl.BlockSpec((TB, 1296), lambda i: (i, 0)),
            pl.BlockSpec((1296, 512), lambda i: (0, 0)),
            pl.BlockSpec((1, 512), lambda i: (0, 0)),
            pl.BlockSpec((512, 512), lambda i: (0, 0)),
            pl.BlockSpec((1, 512), lambda i: (0, 0)),
            pl.BlockSpec((512, 2), lambda i: (0, 0)),
            pl.BlockSpec((1, 2), lambda i: (0, 0)),
        ],
        out_specs=pl.BlockSpec((TB, 2), lambda i: (i, 0)),
        compiler_params=pltpu.CompilerParams(
            dimension_semantics=("parallel",),
            vmem_limit_bytes=48 * 1024 * 1024,
        ),
    )(xf, M, c, fc2_w, fc2_b.reshape(1, 512).astype(f32),
      fc3_w, fc3_b.reshape(1, 2).astype(f32))
    return out[:B] if Bp != B else out


# speedup vs baseline: 1.1303x; 1.1303x over previous
import jax
import jax.numpy as jnp
from jax.experimental import pallas as pl
from jax.experimental.pallas import tpu as pltpu


def _mbuild_kernel(w1t_ref, w2t_ref, b1_ref, b2_ref, w1_ref, fc1b_ref,
                   om_ref, oc_ref, wc_scr, bm_scr, accm_scr, accc_scr):
    f32 = jnp.float32
    h = pl.program_id(0)
    g = pl.program_id(1)

    @pl.when(g == 0)
    def _init():
        wc = jnp.zeros((7, 7, 128), f32)
        full = jnp.zeros((1, 128), f32)
        no_i0 = jnp.zeros((1, 128), f32)
        no_j0 = jnp.zeros((1, 128), f32)
        no_ij = jnp.zeros((1, 128), f32)
        for i in range(3):
            for j in range(3):
                w2ij = w2t_ref[3 * i + j]
                bij = jnp.dot(w1t_ref[...], w2ij,
                              preferred_element_type=f32)
                wc = wc + jnp.pad(bij.reshape(3, 3, 128),
                                  ((2 * i, 4 - 2 * i), (2 * j, 4 - 2 * j),
                                   (0, 0)))
                s2 = jnp.dot(b1_ref[...], w2ij,
                             preferred_element_type=f32)
                full = full + s2
                if i >= 1:
                    no_i0 = no_i0 + s2
                if j >= 1:
                    no_j0 = no_j0 + s2
                if i >= 1 and j >= 1:
                    no_ij = no_ij + s2
        wc_scr[...] = jnp.pad(wc.reshape(49, 128),
                              ((0, 7), (0, 0))).astype(jnp.bfloat16)
        sidx = jax.lax.broadcasted_iota(jnp.int32, (64, 128), 0)
        oh0 = (sidx // 8) == 0
        ow0 = (sidx % 8) == 0
        bm = jnp.where(oh0 & ow0, no_ij,
                       jnp.where(oh0, no_i0, jnp.where(ow0, no_j0, full)))
        bm_scr[...] = bm + b2_ref[...]
        accm_scr[...] = jnp.zeros(accm_scr.shape, f32)
        accc_scr[...] = jnp.where(h == 0, fc1b_ref[...],
                                  jnp.zeros((1, 512), f32))

    w16 = w1_ref[...].astype(jnp.bfloat16)
    part = jnp.dot(wc_scr[...], w16,
                   preferred_element_type=f32)
    contrib = part[:49].reshape(49, 8, 512)
    for a in range(7):
        for b in range(7):
            padded = jnp.pad(contrib[a * 7 + b],
                             ((b // 4, 1 - b // 4), (0, 0)))
            idx = (pl.ds(g + a // 4, 1), a % 4, slice(None), b % 4,
                   slice(None))
            accm_scr[idx] = accm_scr[idx] + padded[None]

    acc = accc_scr[...]
    for sl in range(8):
        row = bm_scr[pl.ds(8 * (4 * h + g) + sl, 1), :]
        acc = acc + jnp.dot(row, w1_ref[:, 512 * sl:512 * (sl + 1)],
                            preferred_element_type=f32)
    accc_scr[...] = acc

    @pl.when(g == 3)
    def _fin():
        om_ref[...] = accm_scr[...].astype(om_ref.dtype)[None]
        oc_ref[...] = accc_scr[...][None]


def _net_kernel(x_ref, om_ref, oc_ref, w2_ref, b2_ref, w3_ref, b3_ref,
                o_ref):
    z = ((0, 0), (0, 0), (0, 0), (0, 0))
    mm = (jnp.pad(om_ref[0], ((0, 4),) + z) +
          jnp.pad(om_ref[1], ((4, 0),) + z))
    m = mm.reshape(1296, 512)
    c = oc_ref[0] + oc_ref[1]
    h = jnp.dot(x_ref[...], m, preferred_element_type=jnp.float32)
    h = jnp.maximum(h + c, 0.0)
    h = jnp.dot(h.astype(jnp.bfloat16), w2_ref[...].astype(jnp.bfloat16),
                preferred_element_type=jnp.float32)
    h = jnp.maximum(h + b2_ref[...], 0.0)
    o = jnp.dot(h.astype(jnp.bfloat16), w3_ref[...].astype(jnp.bfloat16),
                preferred_element_type=jnp.float32)
    o_ref[...] = (o + b3_ref[...]).astype(o_ref.dtype)


def kernel(conv1_w, conv1_b, conv2_w, conv2_b, fc1_w, fc1_b, fc2_w, fc2_b,
           fc3_w, fc3_b, x_nchw):
    f32 = jnp.float32
    bf16 = jnp.bfloat16

    w1t = conv1_w.reshape(64, 9).T
    w2t = conv2_w.reshape(128, 64, 9).transpose(2, 1, 0)

    om, oc = pl.pallas_call(
        _mbuild_kernel,
        out_shape=(jax.ShapeDtypeStruct((2, 5, 4, 9, 4, 512), bf16),
                   jax.ShapeDtypeStruct((2, 1, 512), f32)),
        grid=(2, 4),
        in_specs=[
            pl.BlockSpec((9, 64), lambda h, g: (0, 0)),
            pl.BlockSpec((9, 64, 128), lambda h, g: (0, 0, 0)),
            pl.BlockSpec((1, 64), lambda h, g: (0, 0)),
            pl.BlockSpec((1, 128), lambda h, g: (0, 0)),
            pl.BlockSpec((128, 4096), lambda h, g: (0, 4 * h + g)),
            pl.BlockSpec((1, 512), lambda h, g: (0, 0)),
        ],
        out_specs=(
            pl.BlockSpec((1, 5, 4, 9, 4, 512),
                         lambda h, g: (h, 0, 0, 0, 0, 0)),
            pl.BlockSpec((1, 1, 512), lambda h, g: (h, 0, 0)),
        ),
        scratch_shapes=[
            pltpu.VMEM((56, 128), bf16),
            pltpu.VMEM((64, 128), f32),
            pltpu.VMEM((5, 4, 9, 4, 512), f32),
            pltpu.VMEM((1, 512), f32),
        ],
        compiler_params=pltpu.CompilerParams(
            dimension_semantics=("parallel", "arbitrary"),
            vmem_limit_bytes=64 * 1024 * 1024,
        ),
    )(w1t, w2t, conv1_b.reshape(1, 64), conv2_b.reshape(1, 128),
      fc1_w.reshape(128, 64 * 512), fc1_b.reshape(1, 512))

    B = x_nchw.shape[0]
    xp = jnp.pad(x_nchw.reshape(B, 32, 32).astype(bf16),
                 ((0, 0), (3, 1), (3, 1)))
    xf = xp.reshape(B, 1296)

    TB = 256
    Bp = (B + TB - 1) // TB * TB
    if Bp != B:
        xf = jnp.pad(xf, ((0, Bp - B), (0, 0)))

    out = pl.pallas_call(
        _net_kernel,
        out_shape=jax.ShapeDtypeStruct((Bp, 2), f32),
        grid=(Bp // TB,),
        in_specs=[
            pl.BlockSpec((TB, 1296), lambda i: (i, 0)),
            pl.BlockSpec((2, 5, 4, 9, 4, 512),
                         lambda i: (0, 0, 0, 0, 0, 0)),
            pl.BlockSpec((2, 1, 512), lambda i: (0, 0, 0)),
            pl.BlockSpec((512, 512), lambda i: (0, 0)),
            pl.BlockSpec((1, 512), lambda i: (0, 0)),
            pl.BlockSpec((512, 2), lambda i: (0, 0)),
            pl.BlockSpec((1, 2), lambda i: (0, 0)),
        ],
        out_specs=pl.BlockSpec((TB, 2), lambda i: (i, 0)),
        compiler_params=pltpu.CompilerParams(
            dimension_semantics=("parallel",),
            vmem_limit_bytes=48 * 1024 * 1024,
        ),
    )(xf, om, oc, fc2_w, fc2_b.reshape(1, 512).astype(f32),
      fc3_w, fc3_b.reshape(1, 2).astype(f32))
    return out[:B] if Bp != B else out


# speedup vs baseline: 1.2564x; 1.1115x over previous
import jax
import jax.numpy as jnp
from jax.experimental import pallas as pl
from jax.experimental.pallas import tpu as pltpu


def _mbuild_kernel(w1t_ref, w2t_ref, b1_ref, b2_ref, w1_ref, fc1b_ref,
                   om_ref, oc_ref, wc_scr, bm_scr, accm_scr, accc_scr):
    f32 = jnp.float32
    h = pl.program_id(0)
    g = pl.program_id(1)

    @pl.when(g == 0)
    def _init():
        wc = jnp.zeros((7, 7, 128), f32)
        full = jnp.zeros((1, 128), f32)
        no_i0 = jnp.zeros((1, 128), f32)
        no_j0 = jnp.zeros((1, 128), f32)
        no_ij = jnp.zeros((1, 128), f32)
        for i in range(3):
            for j in range(3):
                w2ij = w2t_ref[3 * i + j]
                bij = jnp.dot(w1t_ref[...], w2ij,
                              preferred_element_type=f32)
                wc = wc + jnp.pad(bij.reshape(3, 3, 128),
                                  ((2 * i, 4 - 2 * i), (2 * j, 4 - 2 * j),
                                   (0, 0)))
                s2 = jnp.dot(b1_ref[...], w2ij,
                             preferred_element_type=f32)
                full = full + s2
                if i >= 1:
                    no_i0 = no_i0 + s2
                if j >= 1:
                    no_j0 = no_j0 + s2
                if i >= 1 and j >= 1:
                    no_ij = no_ij + s2
        wc_scr[...] = jnp.pad(wc.reshape(49, 128),
                              ((0, 7), (0, 0))).astype(jnp.bfloat16)
        sidx = jax.lax.broadcasted_iota(jnp.int32, (64, 128), 0)
        oh0 = (sidx // 8) == 0
        ow0 = (sidx % 8) == 0
        bm = jnp.where(oh0 & ow0, no_ij,
                       jnp.where(oh0, no_i0, jnp.where(ow0, no_j0, full)))
        bm_scr[...] = bm + b2_ref[...]
        accm_scr[...] = jnp.zeros(accm_scr.shape, f32)
        accc_scr[...] = jnp.where(h == 0, fc1b_ref[...],
                                  jnp.zeros((1, 512), f32))

    w16 = w1_ref[...].astype(jnp.bfloat16)
    part = jnp.dot(wc_scr[...], w16,
                   preferred_element_type=f32)
    contrib = part[:49].reshape(49, 8, 512)
    for a in range(7):
        for b in range(7):
            padded = jnp.pad(contrib[a * 7 + b],
                             ((b // 4, 1 - b // 4), (0, 0)))
            idx = (pl.ds(g + a // 4, 1), a % 4, slice(None), b % 4,
                   slice(None))
            accm_scr[idx] = accm_scr[idx] + padded[None]

    acc = accc_scr[...]
    for sl in range(8):
        row = bm_scr[pl.ds(8 * (4 * h + g) + sl, 1), :]
        acc = acc + jnp.dot(row, w1_ref[:, 512 * sl:512 * (sl + 1)],
                            preferred_element_type=f32)
    accc_scr[...] = acc

    @pl.when(g == 3)
    def _fin():
        full = accm_scr[...].reshape(20, 36, 512)
        full = jnp.pad(full, ((0, 2), (0, 0), (0, 0)))
        rows = jnp.where(h == 0, full[3:22], full[0:19])
        om_ref[...] = rows[:, 3:35, :].astype(om_ref.dtype)[None]
        oc_ref[...] = accc_scr[...][None]


def _net_kernel(x_ref, om_ref, oc_ref, w2_ref, b2_ref, w3_ref, b3_ref,
                o_ref):
    z = ((0, 0), (0, 0))
    mm = (jnp.pad(om_ref[0], ((0, 13),) + z) +
          jnp.pad(om_ref[1], ((13, 0),) + z))
    m = mm.reshape(1024, 512)
    c = oc_ref[0] + oc_ref[1]
    h = jnp.dot(x_ref[...].astype(jnp.bfloat16), m,
                preferred_element_type=jnp.float32)
    h = jnp.maximum(h + c, 0.0)
    h = jnp.dot(h.astype(jnp.bfloat16), w2_ref[...].astype(jnp.bfloat16),
                preferred_element_type=jnp.float32)
    h = jnp.maximum(h + b2_ref[...], 0.0)
    o = jnp.dot(h.astype(jnp.bfloat16), w3_ref[...].astype(jnp.bfloat16),
                preferred_element_type=jnp.float32)
    o_ref[...] = (o + b3_ref[...]).astype(o_ref.dtype)


def kernel(conv1_w, conv1_b, conv2_w, conv2_b, fc1_w, fc1_b, fc2_w, fc2_b,
           fc3_w, fc3_b, x_nchw):
    f32 = jnp.float32
    bf16 = jnp.bfloat16

    w1t = conv1_w.reshape(64, 9).T
    w2t = conv2_w.reshape(128, 64, 9).transpose(2, 1, 0)

    om, oc = pl.pallas_call(
        _mbuild_kernel,
        out_shape=(jax.ShapeDtypeStruct((2, 19, 32, 512), bf16),
                   jax.ShapeDtypeStruct((2, 1, 512), f32)),
        grid=(2, 4),
        in_specs=[
            pl.BlockSpec((9, 64), lambda h, g: (0, 0)),
            pl.BlockSpec((9, 64, 128), lambda h, g: (0, 0, 0)),
            pl.BlockSpec((1, 64), lambda h, g: (0, 0)),
            pl.BlockSpec((1, 128), lambda h, g: (0, 0)),
            pl.BlockSpec((128, 4096), lambda h, g: (0, 4 * h + g)),
            pl.BlockSpec((1, 512), lambda h, g: (0, 0)),
        ],
        out_specs=(
            pl.BlockSpec((1, 19, 32, 512), lambda h, g: (h, 0, 0, 0)),
            pl.BlockSpec((1, 1, 512), lambda h, g: (h, 0, 0)),
        ),
        scratch_shapes=[
            pltpu.VMEM((56, 128), bf16),
            pltpu.VMEM((64, 128), f32),
            pltpu.VMEM((5, 4, 9, 4, 512), f32),
            pltpu.VMEM((1, 512), f32),
        ],
        compiler_params=pltpu.CompilerParams(
            dimension_semantics=("parallel", "arbitrary"),
            vmem_limit_bytes=64 * 1024 * 1024,
        ),
    )(w1t, w2t, conv1_b.reshape(1, 64), conv2_b.reshape(1, 128),
      fc1_w.reshape(128, 64 * 512), fc1_b.reshape(1, 512))

    B = x_nchw.shape[0]
    xf = x_nchw.reshape(B, 1024)

    TB = 256
    Bp = (B + TB - 1) // TB * TB
    if Bp != B:
        xf = jnp.pad(xf, ((0, Bp - B), (0, 0)))

    out = pl.pallas_call(
        _net_kernel,
        out_shape=jax.ShapeDtypeStruct((Bp, 2), f32),
        grid=(Bp // TB,),
        in_specs=[
            pl.BlockSpec((TB, 1024), lambda i: (i, 0)),
            pl.BlockSpec((2, 19, 32, 512), lambda i: (0, 0, 0, 0)),
            pl.BlockSpec((2, 1, 512), lambda i: (0, 0, 0)),
            pl.BlockSpec((512, 512), lambda i: (0, 0)),
            pl.BlockSpec((1, 512), lambda i: (0, 0)),
            pl.BlockSpec((512, 2), lambda i: (0, 0)),
            pl.BlockSpec((1, 2), lambda i: (0, 0)),
        ],
        out_specs=pl.BlockSpec((TB, 2), lambda i: (i, 0)),
        compiler_params=pltpu.CompilerParams(
            dimension_semantics=("parallel",),
            vmem_limit_bytes=48 * 1024 * 1024,
        ),
    )(xf, om, oc, fc2_w, fc2_b.reshape(1, 512).astype(f32),
      fc3_w, fc3_b.reshape(1, 2).astype(f32))
    return out[:B] if Bp != B else out


# speedup vs baseline: 1.2866x; 1.0241x over previous
import jax
import jax.numpy as jnp
from jax.experimental import pallas as pl
from jax.experimental.pallas import tpu as pltpu


def _mbuild_kernel(w1t_ref, w2t_ref, b1_ref, b2_ref, w1_ref, fc1b_ref,
                   om_ref, oc_ref, wc_scr, bm_scr, accm_scr, accc_scr):
    f32 = jnp.float32
    h = pl.program_id(0)
    g = pl.program_id(1)

    @pl.when(g == 0)
    def _init():
        wc = jnp.zeros((7, 7, 128), f32)
        full = jnp.zeros((1, 128), f32)
        no_i0 = jnp.zeros((1, 128), f32)
        no_j0 = jnp.zeros((1, 128), f32)
        no_ij = jnp.zeros((1, 128), f32)
        for i in range(3):
            for j in range(3):
                w2ij = w2t_ref[3 * i + j]
                bij = jnp.dot(w1t_ref[...], w2ij,
                              preferred_element_type=f32)
                wc = wc + jnp.pad(bij.reshape(3, 3, 128),
                                  ((2 * i, 4 - 2 * i), (2 * j, 4 - 2 * j),
                                   (0, 0)))
                s2 = jnp.dot(b1_ref[...], w2ij,
                             preferred_element_type=f32)
                full = full + s2
                if i >= 1:
                    no_i0 = no_i0 + s2
                if j >= 1:
                    no_j0 = no_j0 + s2
                if i >= 1 and j >= 1:
                    no_ij = no_ij + s2
        wc_scr[...] = jnp.pad(wc.reshape(49, 128),
                              ((0, 7), (0, 0))).astype(jnp.bfloat16)
        sidx = jax.lax.broadcasted_iota(jnp.int32, (64, 128), 0)
        oh0 = (sidx // 8) == 0
        ow0 = (sidx % 8) == 0
        bm = jnp.where(oh0 & ow0, no_ij,
                       jnp.where(oh0, no_i0, jnp.where(ow0, no_j0, full)))
        bm_scr[...] = bm + b2_ref[...]
        accm_scr[...] = jnp.zeros(accm_scr.shape, f32)
        accc_scr[...] = jnp.where(h == 0, fc1b_ref[...],
                                  jnp.zeros((1, 512), f32))

    w16 = w1_ref[...].astype(jnp.bfloat16)
    part = jnp.dot(wc_scr[...], w16,
                   preferred_element_type=f32)
    contrib = part[:49].reshape(49, 8, 512)
    for a in range(7):
        for b in range(7):
            padded = jnp.pad(contrib[a * 7 + b],
                             ((b // 4, 1 - b // 4), (0, 0)))
            idx = (pl.ds(g + a // 4, 1), a % 4, slice(None), b % 4,
                   slice(None))
            accm_scr[idx] = accm_scr[idx] + padded[None]

    acc = accc_scr[...]
    for sl in range(8):
        row = bm_scr[pl.ds(8 * (4 * h + g) + sl, 1), :]
        acc = acc + jnp.dot(row, w1_ref[:, 512 * sl:512 * (sl + 1)],
                            preferred_element_type=f32)
    accc_scr[...] = acc

    @pl.when(g == 3)
    def _fin():
        full = accm_scr[...].reshape(20, 36, 512)
        full = jnp.pad(full, ((0, 2), (0, 0), (0, 0)))
        rows = jnp.where(h == 0, full[3:22], full[0:19])
        om_ref[...] = rows[:, 3:35, :].astype(om_ref.dtype)[None]
        oc_ref[...] = accc_scr[...][None]


def _net_kernel(x_ref, om_ref, oc_ref, w2_ref, b2_ref, w3_ref, b3_ref,
                o_ref):
    z = ((0, 0), (0, 0))
    mm = (jnp.pad(om_ref[0], ((0, 13),) + z) +
          jnp.pad(om_ref[1], ((13, 0),) + z))
    m = mm.reshape(1024, 512)
    c = oc_ref[0] + oc_ref[1]
    h = jnp.dot(x_ref[...].astype(jnp.bfloat16), m,
                preferred_element_type=jnp.float32)
    h = jnp.maximum(h + c, 0.0)
    h = jnp.dot(h.astype(jnp.bfloat16), w2_ref[...].astype(jnp.bfloat16),
                preferred_element_type=jnp.float32)
    h = jnp.maximum(h + b2_ref[...], 0.0)
    o = jnp.dot(h.astype(jnp.bfloat16), w3_ref[...].astype(jnp.bfloat16),
                preferred_element_type=jnp.float32)
    o_ref[...] = (o + b3_ref[...]).astype(o_ref.dtype)


def kernel(conv1_w, conv1_b, conv2_w, conv2_b, fc1_w, fc1_b, fc2_w, fc2_b,
           fc3_w, fc3_b, x_nchw):
    f32 = jnp.float32
    bf16 = jnp.bfloat16

    w1t = conv1_w.reshape(64, 9).T
    w2t = conv2_w.reshape(128, 64, 9).transpose(2, 1, 0)

    om, oc = pl.pallas_call(
        _mbuild_kernel,
        out_shape=(jax.ShapeDtypeStruct((2, 19, 32, 512), bf16),
                   jax.ShapeDtypeStruct((2, 1, 512), f32)),
        grid=(2, 4),
        in_specs=[
            pl.BlockSpec((9, 64), lambda h, g: (0, 0)),
            pl.BlockSpec((9, 64, 128), lambda h, g: (0, 0, 0)),
            pl.BlockSpec((1, 64), lambda h, g: (0, 0)),
            pl.BlockSpec((1, 128), lambda h, g: (0, 0)),
            pl.BlockSpec((128, 4096), lambda h, g: (0, 4 * h + g)),
            pl.BlockSpec((1, 512), lambda h, g: (0, 0)),
        ],
        out_specs=(
            pl.BlockSpec((1, 19, 32, 512), lambda h, g: (h, 0, 0, 0)),
            pl.BlockSpec((1, 1, 512), lambda h, g: (h, 0, 0)),
        ),
        scratch_shapes=[
            pltpu.VMEM((56, 128), bf16),
            pltpu.VMEM((64, 128), f32),
            pltpu.VMEM((5, 4, 9, 4, 512), f32),
            pltpu.VMEM((1, 512), f32),
        ],
        compiler_params=pltpu.CompilerParams(
            dimension_semantics=("parallel", "arbitrary"),
            vmem_limit_bytes=64 * 1024 * 1024,
        ),
    )(w1t, w2t, conv1_b.reshape(1, 64), conv2_b.reshape(1, 128),
      fc1_w.reshape(128, 64 * 512), fc1_b.reshape(1, 512))

    B = x_nchw.shape[0]
    xf = x_nchw.reshape(B, 1024)

    TB = 512 if B % 512 == 0 else 128
    Bp = (B + TB - 1) // TB * TB
    if Bp != B:
        xf = jnp.pad(xf, ((0, Bp - B), (0, 0)))

    out = pl.pallas_call(
        _net_kernel,
        out_shape=jax.ShapeDtypeStruct((Bp, 2), f32),
        grid=(Bp // TB,),
        in_specs=[
            pl.BlockSpec((TB, 1024), lambda i: (i, 0)),
            pl.BlockSpec((2, 19, 32, 512), lambda i: (0, 0, 0, 0)),
            pl.BlockSpec((2, 1, 512), lambda i: (0, 0, 0)),
            pl.BlockSpec((512, 512), lambda i: (0, 0)),
            pl.BlockSpec((1, 512), lambda i: (0, 0)),
            pl.BlockSpec((512, 2), lambda i: (0, 0)),
            pl.BlockSpec((1, 2), lambda i: (0, 0)),
        ],
        out_specs=pl.BlockSpec((TB, 2), lambda i: (i, 0)),
        compiler_params=pltpu.CompilerParams(
            dimension_semantics=("parallel",),
            vmem_limit_bytes=48 * 1024 * 1024,
        ),
    )(xf, om, oc, fc2_w, fc2_b.reshape(1, 512).astype(f32),
      fc3_w, fc3_b.reshape(1, 2).astype(f32))
    return out[:B] if Bp != B else out
